# baseline (speedup 1.0000x reference)
import jax
import jax.numpy as jnp
from jax.experimental import pallas as pl
from jax.experimental.pallas import tpu as pltpu

QUEUE_SIZE = 131072
FEATURE_DIM = 128
BATCH = 16384

BLOCK_ROWS = 8192
GRID = QUEUE_SIZE // BLOCK_ROWS
FEAT_BLOCKS = BATCH // BLOCK_ROWS

LBL_COLS = 128
LBL_ROWS_Q = QUEUE_SIZE // LBL_COLS
LBL_ROWS_B = BATCH // LBL_COLS
LBL_BLOCK = BLOCK_ROWS // LBL_COLS


def _copy_kernel(features, feat, true2d, pred2d, tl2d, pl2d,
                 out_f, out_t, out_p):
    i = pl.program_id(0)

    @pl.when(i < FEAT_BLOCKS)
    def _():
        out_f[...] = feat[...]
        out_t[...] = true2d[...]
        out_p[...] = pred2d[...]

    @pl.when(i >= FEAT_BLOCKS)
    def _():
        out_f[...] = features[...]
        out_t[...] = tl2d[...]
        out_p[...] = pl2d[...]


def kernel(feat, true, pred, features, true_labels, pred_labels):
    true2d = true.reshape(LBL_ROWS_B, LBL_COLS)
    pred2d = pred.reshape(LBL_ROWS_B, LBL_COLS)
    tl2d = true_labels.reshape(LBL_ROWS_Q, LBL_COLS)
    pl2d = pred_labels.reshape(LBL_ROWS_Q, LBL_COLS)

    new_idx = lambda i: (jnp.minimum(i, FEAT_BLOCKS - 1), 0)
    old_idx = lambda i: (jnp.maximum(i, FEAT_BLOCKS), 0)

    out_f, out_t, out_p = pl.pallas_call(
        _copy_kernel,
        grid=(GRID,),
        in_specs=[
            pl.BlockSpec((BLOCK_ROWS, FEATURE_DIM), old_idx),
            pl.BlockSpec((BLOCK_ROWS, FEATURE_DIM), new_idx),
            pl.BlockSpec((LBL_BLOCK, LBL_COLS), new_idx),
            pl.BlockSpec((LBL_BLOCK, LBL_COLS), new_idx),
            pl.BlockSpec((LBL_BLOCK, LBL_COLS), old_idx),
            pl.BlockSpec((LBL_BLOCK, LBL_COLS), old_idx),
        ],
        out_specs=[
            pl.BlockSpec((BLOCK_ROWS, FEATURE_DIM), lambda i: (i, 0)),
            pl.BlockSpec((LBL_BLOCK, LBL_COLS), lambda i: (i, 0)),
            pl.BlockSpec((LBL_BLOCK, LBL_COLS), lambda i: (i, 0)),
        ],
        out_shape=[
            jax.ShapeDtypeStruct((QUEUE_SIZE, FEATURE_DIM), jnp.float32),
            jax.ShapeDtypeStruct((LBL_ROWS_Q, LBL_COLS), jnp.int32),
            jax.ShapeDtypeStruct((LBL_ROWS_Q, LBL_COLS), jnp.int32),
        ],
        compiler_params=pltpu.CompilerParams(
            dimension_semantics=("parallel",),
        ),
    )(features, feat, true2d, pred2d, tl2d, pl2d)

    return (out_f, out_t.reshape(QUEUE_SIZE), out_p.reshape(QUEUE_SIZE))
